# baseline (speedup 1.0000x reference)
import functools

import jax
import jax.numpy as jnp
from jax.experimental import pallas as pl
from jax.experimental.pallas import tpu as pltpu


def _gcn_body(nstrip, x_ref, adj_ref, w_ref, b_ref, out_ref, adjb_s, cs8_s):
    i = pl.program_id(0)
    n = adjb_s.shape[0]
    rows = n // nstrip

    strip = adj_ref[:].astype(jnp.bfloat16)
    adjb_s[pl.ds(i * rows, rows), :] = strip
    ones8 = jnp.full((8, rows), 1.0, dtype=jnp.bfloat16)
    part = jax.lax.dot_general(ones8, strip, (((1,), (0,)), ((), ())),
                               preferred_element_type=jnp.float32)

    @pl.when(i == 0)
    def _init():
        cs8_s[:] = part

    @pl.when(i > 0)
    def _acc():
        cs8_s[:] = cs8_s[:] + part

    @pl.when(i == nstrip - 1)
    def _tail():
        colsum = cs8_s[0:1, :]
        dinv = jax.lax.rsqrt(2.0 * colsum + 1.0)
        ht = jax.lax.dot_general(w_ref[:], x_ref[:], (((0,), (1,)), ((), ())),
                                 preferred_element_type=jnp.float32)
        hht = ht * dinv
        hi = hht.astype(jnp.bfloat16)
        lo = (hht - hi.astype(jnp.float32)).astype(jnp.bfloat16)
        adjb = adjb_s[:]
        st = (jax.lax.dot_general(hi, adjb, (((1,), (0,)), ((), ())),
                                  preferred_element_type=jnp.float32) +
              jax.lax.dot_general(lo, adjb, (((1,), (0,)), ((), ())),
                                  preferred_element_type=jnp.float32))
        out_ref[:] = jnp.tanh(dinv * (2.0 * st + hht) + b_ref[:])


def kernel(x, adj, W, b):
    n, in_c = x.shape
    out_c = W.shape[1]
    nstrip = 8
    rows = n // nstrip
    body = functools.partial(_gcn_body, nstrip)
    return pl.pallas_call(
        body,
        grid=(nstrip,),
        in_specs=[
            pl.BlockSpec((n, in_c), lambda i: (0, 0)),
            pl.BlockSpec((rows, n), lambda i: (i, 0)),
            pl.BlockSpec((in_c, out_c), lambda i: (0, 0)),
            pl.BlockSpec((out_c, 1), lambda i: (0, 0)),
        ],
        out_specs=pl.BlockSpec((out_c, n), lambda i: (0, 0)),
        out_shape=jax.ShapeDtypeStruct((out_c, n), jnp.float32),
        scratch_shapes=[
            pltpu.VMEM((n, n), jnp.bfloat16),
            pltpu.VMEM((8, n), jnp.float32),
        ],
    )(x, adj, W, b.reshape(out_c, 1))


# speedup vs baseline: 1.1120x; 1.1120x over previous
import functools

import jax
import jax.numpy as jnp
from jax.experimental import pallas as pl
from jax.experimental.pallas import tpu as pltpu


def _gcn_body(nstrip, x_hbm, adj_hbm, w_hbm, b_hbm, out_ref,
              x_s, adj_s, w_s, b_s, sems, sem_x, sem_w, sem_b):
    n = adj_s.shape[0]
    rows = n // nstrip
    strip_copy = [
        pltpu.make_async_copy(
            adj_hbm.at[pl.ds(i * rows, rows), :],
            adj_s.at[pl.ds(i * rows, rows), :],
            sems.at[i],
        )
        for i in range(nstrip)
    ]
    cx = pltpu.make_async_copy(x_hbm, x_s, sem_x)
    cw = pltpu.make_async_copy(w_hbm, w_s, sem_w)
    cb = pltpu.make_async_copy(b_hbm, b_s, sem_b)
    strip_copy[0].start()
    strip_copy[1].start()
    cx.start()
    cw.start()
    cb.start()
    cx.wait()
    cw.wait()
    ht = jax.lax.dot_general(w_s[:], x_s[:], (((0,), (1,)), ((), ())),
                             preferred_element_type=jnp.float32)
    colsum = jnp.zeros((1, n), dtype=jnp.float32)
    for i in range(nstrip):
        strip_copy[i].wait()
        if i + 2 < nstrip:
            strip_copy[i + 2].start()
        colsum = colsum + jnp.sum(adj_s[i * rows:(i + 1) * rows, :],
                                  axis=0, keepdims=True)
    dinv = jax.lax.rsqrt(2.0 * colsum + 1.0)
    hht = ht * dinv
    st = jax.lax.dot_general(hht, adj_s[:], (((1,), (0,)), ((), ())),
                             preferred_element_type=jnp.float32)
    cb.wait()
    out_ref[:] = jnp.tanh(dinv * (2.0 * st + hht) + b_s[:])


def kernel(x, adj, W, b):
    n, in_c = x.shape
    out_c = W.shape[1]
    nstrip = 8
    body = functools.partial(_gcn_body, nstrip)
    hbm = pl.BlockSpec(memory_space=pltpu.MemorySpace.HBM)
    return pl.pallas_call(
        body,
        in_specs=[hbm, hbm, hbm, hbm],
        out_specs=pl.BlockSpec((out_c, n), lambda: (0, 0)),
        out_shape=jax.ShapeDtypeStruct((out_c, n), jnp.float32),
        scratch_shapes=[
            pltpu.VMEM((n, in_c), jnp.float32),
            pltpu.VMEM((n, n), jnp.float32),
            pltpu.VMEM((in_c, out_c), jnp.float32),
            pltpu.VMEM((out_c, 1), jnp.float32),
            pltpu.SemaphoreType.DMA((nstrip,)),
            pltpu.SemaphoreType.DMA,
            pltpu.SemaphoreType.DMA,
            pltpu.SemaphoreType.DMA,
        ],
    )(x, adj, W, b.reshape(out_c, 1))


# speedup vs baseline: 1.2270x; 1.1035x over previous
import functools

import jax
import jax.numpy as jnp
from jax.experimental import pallas as pl
from jax.experimental.pallas import tpu as pltpu


def _gcn_body(nstrip, x_hbm, adj_hbm, w_hbm, b_hbm, out_ref,
              x_s, adj_s, w_s, b_s, sems, sem_x, sem_w, sem_b):
    n = adj_s.shape[0]
    rows = n // nstrip
    strip_copy = [
        pltpu.make_async_copy(
            adj_hbm.at[pl.ds(i * rows, rows), :],
            adj_s.at[pl.ds(i * rows, rows), :],
            sems.at[i],
        )
        for i in range(nstrip)
    ]
    cx = pltpu.make_async_copy(x_hbm, x_s, sem_x)
    cw = pltpu.make_async_copy(w_hbm, w_s, sem_w)
    cb = pltpu.make_async_copy(b_hbm, b_s, sem_b)
    for c in strip_copy:
        c.start()
    cx.start()
    cw.start()
    cb.start()
    cx.wait()
    cw.wait()
    ht = jax.lax.dot_general(w_s[:], x_s[:], (((0,), (1,)), ((), ())),
                             preferred_element_type=jnp.float32)
    colsum = jnp.zeros((1, n), dtype=jnp.float32)
    for i in range(nstrip):
        strip_copy[i].wait()
        colsum = colsum + jnp.sum(adj_s[i * rows:(i + 1) * rows, :],
                                  axis=0, keepdims=True)
    dinv = jax.lax.rsqrt(2.0 * colsum + 1.0)
    hht = ht * dinv
    st = jax.lax.dot_general(hht, adj_s[:], (((1,), (0,)), ((), ())),
                             preferred_element_type=jnp.float32)
    cb.wait()
    out_ref[:] = jnp.tanh(dinv * (2.0 * st + hht) + b_s[:])


def kernel(x, adj, W, b):
    n, in_c = x.shape
    out_c = W.shape[1]
    nstrip = 8
    body = functools.partial(_gcn_body, nstrip)
    hbm = pl.BlockSpec(memory_space=pltpu.MemorySpace.HBM)
    return pl.pallas_call(
        body,
        in_specs=[hbm, hbm, hbm, hbm],
        out_specs=pl.BlockSpec((out_c, n), lambda: (0, 0)),
        out_shape=jax.ShapeDtypeStruct((out_c, n), jnp.float32),
        scratch_shapes=[
            pltpu.VMEM((n, in_c), jnp.float32),
            pltpu.VMEM((n, n), jnp.float32),
            pltpu.VMEM((in_c, out_c), jnp.float32),
            pltpu.VMEM((out_c, 1), jnp.float32),
            pltpu.SemaphoreType.DMA((nstrip,)),
            pltpu.SemaphoreType.DMA,
            pltpu.SemaphoreType.DMA,
            pltpu.SemaphoreType.DMA,
        ],
    )(x, adj, W, b.reshape(out_c, 1))


# speedup vs baseline: 1.2569x; 1.0243x over previous
import jax
import jax.numpy as jnp
from jax.experimental import pallas as pl


def _gcn_body(x_ref, adj_ref, w_ref, b_ref, out_ref):
    adj = adj_ref[:]
    colsum = jnp.sum(adj, axis=0, keepdims=True)
    dinv = jax.lax.rsqrt(2.0 * colsum + 1.0)
    ht = jax.lax.dot_general(w_ref[:], x_ref[:], (((0,), (1,)), ((), ())),
                             preferred_element_type=jnp.float32)
    hht = ht * dinv
    st = jnp.dot(hht, adj, preferred_element_type=jnp.float32)
    outt = dinv * (2.0 * st + hht) + b_ref[:]
    out_ref[:] = jnp.tanh(outt)


def kernel(x, adj, W, b):
    n = x.shape[0]
    out_c = W.shape[1]
    return pl.pallas_call(
        _gcn_body,
        out_shape=jax.ShapeDtypeStruct((out_c, n), jnp.float32),
    )(x, adj, W, b.reshape(out_c, 1))
